# Initial kernel scaffold; baseline (speedup 1.0000x reference)
import jax
import jax.numpy as jnp
from jax.experimental import pallas as pl


def kernel(x, edge_index, batch, W1, b1, W2, b2):
    raise NotImplementedError("write your pallas kernel here")


# speedup vs baseline: 12.8227x; 12.8227x over previous
import functools

import jax
import jax.numpy as jnp
from jax import lax
from jax.experimental import pallas as pl
from jax.experimental.pallas import tpu as pltpu
from jax.experimental.pallas import tpu_sc as plsc

N = 10000
E = 320000
D = 128
G = 64

NC = 2
NS = 16
NW = NC * NS
C = 128
CH = 79
EPT = CH * C
EP = NW * EPT
NP = 10240
RPT = NP // NS
DEGW = 128
RB = 1024
NB = NP // RB

_sc_mesh = plsc.VectorSubcoreMesh(core_axis_name="c", subcore_axis_name="s")


@functools.partial(
    pl.kernel,
    out_type=jax.ShapeDtypeStruct((NC, NP, DEGW), jnp.float32),
    mesh=_sc_mesh,
    scratch_types=[
        pltpu.VMEM((CH, C), jnp.int32),
        pltpu.VMEM((C, DEGW), jnp.float32),
        pltpu.VMEM_SHARED((NP, DEGW), jnp.float32),
    ],
)
def _sc_degree(dst3, ones, zeros, out, dst_v, ones_v, acc):
    c = lax.axis_index("c")
    s = lax.axis_index("s")
    wid = c * NS + s
    pltpu.sync_copy(dst3.at[wid], dst_v)
    pltpu.sync_copy(ones, ones_v)
    pltpu.sync_copy(zeros, acc.at[pl.ds(s * RPT, RPT)])
    plsc.subcore_barrier()

    def body(j, carry):
        pltpu.sync_copy(ones_v, acc.at[dst_v.at[j]], add=True)
        return carry

    lax.fori_loop(0, CH, body, 0)
    plsc.subcore_barrier()
    pltpu.sync_copy(acc.at[pl.ds(s * RPT, RPT)], out.at[c, pl.ds(s * RPT, RPT)])


@functools.partial(
    pl.kernel,
    out_type=jax.ShapeDtypeStruct((NC, NP, D), jnp.float32),
    mesh=_sc_mesh,
    scratch_types=[
        pltpu.VMEM((CH, C), jnp.int32),
        pltpu.VMEM((CH, C), jnp.int32),
        pltpu.VMEM((C, D), jnp.float32),
        pltpu.VMEM_SHARED((NP, D), jnp.float32),
        pltpu.SemaphoreType.DMA,
    ],
)
def _sc_edge_agg(table, src3, dst3, zeros, out, src_v, dst_v, rows_v, acc, sem):
    c = lax.axis_index("c")
    s = lax.axis_index("s")
    wid = c * NS + s
    pltpu.sync_copy(src3.at[wid], src_v)
    pltpu.sync_copy(dst3.at[wid], dst_v)
    pltpu.sync_copy(zeros, acc.at[pl.ds(s * RPT, RPT)])
    plsc.subcore_barrier()

    def body(j, carry):
        pltpu.async_copy(table.at[src_v.at[j]], rows_v, sem).wait()
        pltpu.sync_copy(rows_v, acc.at[dst_v.at[j]], add=True)
        return carry

    lax.fori_loop(0, CH, body, 0)
    plsc.subcore_barrier()
    pltpu.sync_copy(acc.at[pl.ds(s * RPT, RPT)], out.at[c, pl.ds(s * RPT, RPT)])


def _dinv(degp_ref):
    deg = degp_ref[0, :, 0:1] + degp_ref[1, :, 0:1] + 1.0
    return lax.rsqrt(deg)


def _mm_scale_body(x_ref, w_ref, degp_ref, o_ref):
    h = jnp.dot(x_ref[...], w_ref[...], preferred_element_type=jnp.float32)
    o_ref[...] = h * _dinv(degp_ref)


def _mm_scale(x, w, degp):
    return pl.pallas_call(
        _mm_scale_body,
        grid=(NB,),
        in_specs=[
            pl.BlockSpec((RB, D), lambda i: (i, 0)),
            pl.BlockSpec((D, D), lambda i: (0, 0)),
            pl.BlockSpec((NC, RB, DEGW), lambda i: (0, i, 0)),
        ],
        out_specs=pl.BlockSpec((RB, D), lambda i: (i, 0)),
        out_shape=jax.ShapeDtypeStruct((NP, D), jnp.float32),
    )(x, w, degp)


def _combine_mm_body(parts_ref, hs_ref, degp_ref, b_ref, w_ref, o_ref):
    dinv = _dinv(degp_ref)
    agg = parts_ref[0] + parts_ref[1] + hs_ref[...]
    z = jnp.maximum(agg * dinv + b_ref[...], 0.0)
    o_ref[...] = jnp.dot(z, w_ref[...], preferred_element_type=jnp.float32) * dinv


def _combine_mm(parts, hs, degp, b, w):
    return pl.pallas_call(
        _combine_mm_body,
        grid=(NB,),
        in_specs=[
            pl.BlockSpec((NC, RB, D), lambda i: (0, i, 0)),
            pl.BlockSpec((RB, D), lambda i: (i, 0)),
            pl.BlockSpec((NC, RB, DEGW), lambda i: (0, i, 0)),
            pl.BlockSpec((1, D), lambda i: (0, 0)),
            pl.BlockSpec((D, D), lambda i: (0, 0)),
        ],
        out_specs=pl.BlockSpec((RB, D), lambda i: (i, 0)),
        out_shape=jax.ShapeDtypeStruct((NP, D), jnp.float32),
    )(parts, hs, degp, b, w)


def _combine_pool_body(parts_ref, hs_ref, degp_ref, b_ref, bid_ref, o_ref,
                       sum_sc, cnt_sc):
    i = pl.program_id(0)

    @pl.when(i == 0)
    def _():
        sum_sc[...] = jnp.zeros_like(sum_sc)
        cnt_sc[...] = jnp.zeros_like(cnt_sc)

    dinv = _dinv(degp_ref)
    agg = parts_ref[0] + parts_ref[1] + hs_ref[...]
    z = jnp.maximum(agg * dinv + b_ref[...], 0.0)
    oh = (bid_ref[...] == lax.broadcasted_iota(jnp.int32, (RB, G), 1))
    oh = oh.astype(jnp.float32)
    sum_sc[...] += lax.dot_general(oh, z, (((0,), (0,)), ((), ())),
                                   preferred_element_type=jnp.float32)
    cnt_sc[...] += jnp.broadcast_to(jnp.sum(oh, axis=0)[:, None], (G, D))

    @pl.when(i == NB - 1)
    def _():
        o_ref[...] = sum_sc[...] / jnp.maximum(cnt_sc[...], 1.0)


def _combine_pool(parts, hs, degp, b, bid):
    return pl.pallas_call(
        _combine_pool_body,
        grid=(NB,),
        in_specs=[
            pl.BlockSpec((NC, RB, D), lambda i: (0, i, 0)),
            pl.BlockSpec((RB, D), lambda i: (i, 0)),
            pl.BlockSpec((NC, RB, DEGW), lambda i: (0, i, 0)),
            pl.BlockSpec((1, D), lambda i: (0, 0)),
            pl.BlockSpec((RB, G), lambda i: (i, 0)),
        ],
        out_specs=pl.BlockSpec((G, D), lambda i: (0, 0)),
        out_shape=jax.ShapeDtypeStruct((G, D), jnp.float32),
        scratch_shapes=[
            pltpu.VMEM((G, D), jnp.float32),
            pltpu.VMEM((G, D), jnp.float32),
        ],
    )(parts, hs, degp, b, bid)


def kernel(x, edge_index, batch, W1, b1, W2, b2):
    src = edge_index[0]
    dst = edge_index[1]
    pad_e = EP - E
    srcp = jnp.concatenate(
        [src, jnp.full((pad_e,), N, jnp.int32)]).reshape(NW, CH, C)
    dstp = jnp.concatenate(
        [dst, jnp.full((pad_e,), N, jnp.int32)]).reshape(NW, CH, C)
    xp = jnp.pad(x, ((0, NP - N), (0, 0)))
    zeros_agg = jnp.zeros((RPT, D), jnp.float32)
    zeros_deg = jnp.zeros((RPT, DEGW), jnp.float32)
    ones_deg = jnp.ones((C, DEGW), jnp.float32)
    batch_pad = jnp.concatenate([batch, jnp.full((NP - N,), G, jnp.int32)])
    bid = jnp.broadcast_to(batch_pad[:, None], (NP, G))

    degp = _sc_degree(dstp, ones_deg, zeros_deg)
    hs1 = _mm_scale(xp, W1, degp)
    parts1 = _sc_edge_agg(hs1, srcp, dstp, zeros_agg)
    hs2 = _combine_mm(parts1, hs1, degp, b1.reshape(1, D), W2)
    parts2 = _sc_edge_agg(hs2, srcp, dstp, zeros_agg)
    pooled = _combine_pool(parts2, hs2, degp, b2.reshape(1, D), bid)
    return pooled
